# Initial kernel scaffold; baseline (speedup 1.0000x reference)
import jax
import jax.numpy as jnp
from jax.experimental import pallas as pl


def kernel(x, angle):
    raise NotImplementedError("write your pallas kernel here")


# speedup vs baseline: 1.5101x; 1.5101x over previous
import jax
import jax.numpy as jnp
from jax.experimental import pallas as pl
from jax.experimental.pallas import tpu as pltpu

ROWS = 59049
BATCH = 128
SUB = 243
GROUP = 3 * SUB
BLK_GROUPS = 8
BLK = GROUP * BLK_GROUPS


def _rot_kernel(ang_ref, x_ref, re_ref, im_ref):
    half = 0.5 * ang_ref[0]
    c = jnp.cos(half)
    ns = -jnp.sin(half)
    for g in range(BLK_GROUPS):
        b0 = g * GROUP
        x0 = x_ref[b0:b0 + SUB, :]
        x1 = x_ref[b0 + SUB:b0 + 2 * SUB, :]
        x2 = x_ref[b0 + 2 * SUB:b0 + 3 * SUB, :]
        re_ref[b0:b0 + SUB, :] = c * x0
        re_ref[b0 + SUB:b0 + 2 * SUB, :] = c * x1
        re_ref[b0 + 2 * SUB:b0 + 3 * SUB, :] = x2
        im_ref[b0:b0 + SUB, :] = ns * x1
        im_ref[b0 + SUB:b0 + 2 * SUB, :] = ns * x0
        im_ref[b0 + 2 * SUB:b0 + 3 * SUB, :] = jnp.zeros_like(x2)


def kernel(x, angle):
    grid = (pl.cdiv(ROWS, BLK),)
    re, im = pl.pallas_call(
        _rot_kernel,
        grid=grid,
        in_specs=[
            pl.BlockSpec(memory_space=pltpu.SMEM),
            pl.BlockSpec((BLK, BATCH), lambda t: (t, 0)),
        ],
        out_specs=[
            pl.BlockSpec((BLK, BATCH), lambda t: (t, 0)),
            pl.BlockSpec((BLK, BATCH), lambda t: (t, 0)),
        ],
        out_shape=[
            jax.ShapeDtypeStruct((ROWS, BATCH), jnp.float32),
            jax.ShapeDtypeStruct((ROWS, BATCH), jnp.float32),
        ],
    )(angle, x)
    return jax.lax.complex(re, im)
